# baseline (speedup 1.0000x reference)
import functools
import math

import jax
import jax.numpy as jnp
from jax import lax
from jax.experimental import pallas as pl
from jax.experimental.pallas import tpu as pltpu
from jax.experimental.pallas import tpu_sc as plsc

VOCAB = 100000
SEQ_LEN = 2048
D_MODEL = 768
BATCH = 4

NUM_WORKERS = 32
POS_PER_W = SEQ_LEN // NUM_WORKERS
CHUNK = 32
NCHUNKS = BATCH * POS_PER_W // CHUNK
NBUF = 3
LANES = 16
GROUPS = D_MODEL // LANES
SCALE = math.sqrt(float(D_MODEL))


def _body(idx_hbm, tok_hbm, pos_hbm, out_hbm,
          idx0, idx1, idx2, tok0, tok1, tok2, pos_v,
          g0, g1, g2, s0, s1, s2):
    idx_v = [idx0, idx1, idx2]
    tok_v = [tok0, tok1, tok2]
    gsem = [g0, g1, g2]
    ssem = [s0, s1, s2]

    wid = lax.axis_index("s") * 2 + lax.axis_index("c")
    pos_base = wid * POS_PER_W

    pltpu.sync_copy(pos_hbm.at[pl.ds(pos_base, POS_PER_W)], pos_v)

    def off(k):
        b, h = divmod(k, 2)
        return b * SEQ_LEN + pos_base + h * CHUNK

    def start_gather(k):
        p = k % NBUF
        pltpu.sync_copy(idx_hbm.at[pl.ds(off(k), CHUNK)], idx_v[p])
        return pltpu.async_copy(tok_hbm.at[idx_v[p]], tok_v[p], gsem[p])

    gathers = {}
    stores = {}
    for k in range(NBUF - 1):
        gathers[k] = start_gather(k)

    for k in range(NCHUNKS):
        p = k % NBUF
        gathers[k].wait()
        h = k % 2

        def row(r, _):
            for j in range(GROUPS):
                sl = pl.ds(j * LANES, LANES)
                tok_v[p][r, sl] = (tok_v[p][r, sl] * SCALE
                                   + pos_v[h * CHUNK + r, sl])
            return _

        lax.fori_loop(0, CHUNK, row, 0)
        stores[k] = pltpu.async_copy(tok_v[p], out_hbm.at[pl.ds(off(k), CHUNK)],
                                     ssem[p])
        nxt = k + NBUF - 1
        if nxt < NCHUNKS:
            q = nxt % NBUF
            if nxt - NBUF >= 0:
                stores[nxt - NBUF].wait()
            gathers[nxt] = start_gather(nxt)

    for k in range(NCHUNKS - NBUF, NCHUNKS):
        stores[k].wait()


@jax.jit
def _embed(idx_flat, tok_table, pos_table):
    mesh = plsc.VectorSubcoreMesh(core_axis_name="c", subcore_axis_name="s")
    k = functools.partial(
        pl.kernel,
        mesh=mesh,
        out_type=jax.ShapeDtypeStruct((BATCH * SEQ_LEN, D_MODEL), jnp.float32),
        scratch_types=(
            [pltpu.VMEM((CHUNK,), jnp.int32) for _ in range(NBUF)]
            + [pltpu.VMEM((CHUNK, D_MODEL), jnp.float32) for _ in range(NBUF)]
            + [pltpu.VMEM((POS_PER_W, D_MODEL), jnp.float32)]
            + [pltpu.SemaphoreType.DMA for _ in range(2 * NBUF)]
        ),
    )(_body)
    return k(idx_flat, tok_table, pos_table)


def kernel(inputs, tok_table, pos_table):
    idx_flat = inputs.astype(jnp.int32).reshape(-1)
    out = _embed(idx_flat, tok_table, pos_table)
    return out.reshape(BATCH, SEQ_LEN, D_MODEL)


# speedup vs baseline: 1.0365x; 1.0365x over previous
import functools
import math

import jax
import jax.numpy as jnp
from jax import lax
from jax.experimental import pallas as pl
from jax.experimental.pallas import tpu as pltpu
from jax.experimental.pallas import tpu_sc as plsc

VOCAB = 100000
SEQ_LEN = 2048
D_MODEL = 768
BATCH = 4

NUM_WORKERS = 32
POS_PER_W = SEQ_LEN // NUM_WORKERS
CHUNK = 32
NCHUNKS = BATCH * POS_PER_W // CHUNK
NBUF = 3
LANES = 16
GROUPS = D_MODEL // LANES
SCALE = math.sqrt(float(D_MODEL))


def _body(idx_hbm, tok_hbm, pos_hbm, out_hbm,
          idx_v, tok0, tok1, tok2, pos_v,
          isem, psem, g0, g1, g2, s0, s1, s2):
    tok_v = [tok0, tok1, tok2]
    gsem = [g0, g1, g2]
    ssem = [s0, s1, s2]

    wid = lax.axis_index("s") * 2 + lax.axis_index("c")
    pos_base = wid * POS_PER_W

    pos_cp = pltpu.async_copy(pos_hbm.at[pl.ds(pos_base, POS_PER_W)],
                              pos_v, psem)
    idx_cps = []
    for b in range(BATCH):
        idx_cps.append(pltpu.async_copy(
            idx_hbm.at[b, pl.ds(pos_base, POS_PER_W)],
            idx_v.at[pl.ds(b * POS_PER_W, POS_PER_W)], isem))
    for cp in idx_cps:
        cp.wait()

    def start_gather(k):
        p = k % NBUF
        return pltpu.async_copy(
            tok_hbm.at[idx_v.at[pl.ds(k * CHUNK, CHUNK)]], tok_v[p], gsem[p])

    gathers = {}
    stores = {}
    for k in range(NBUF - 1):
        gathers[k] = start_gather(k)
    pos_cp.wait()

    for k in range(NCHUNKS):
        p = k % NBUF
        b, h = divmod(k, 2)
        gathers[k].wait()

        def row(r, _):
            for j in range(GROUPS):
                sl = pl.ds(j * LANES, LANES)
                tok_v[p][r, sl] = (tok_v[p][r, sl] * SCALE
                                   + pos_v[h * CHUNK + r, sl])
            return _

        lax.fori_loop(0, CHUNK, row, 0)
        nxt = k + NBUF - 1
        if nxt < NCHUNKS:
            if nxt - NBUF >= 0:
                stores[nxt - NBUF].wait()
            gathers[nxt] = start_gather(nxt)
        stores[k] = pltpu.async_copy(
            tok_v[p],
            out_hbm.at[b, pl.ds(pos_base + h * CHUNK, CHUNK)], ssem[p])

    for k in range(NCHUNKS - NBUF, NCHUNKS):
        stores[k].wait()


@jax.jit
def _embed(idx, tok_table, pos_table):
    mesh = plsc.VectorSubcoreMesh(core_axis_name="c", subcore_axis_name="s")
    k = functools.partial(
        pl.kernel,
        mesh=mesh,
        out_type=jax.ShapeDtypeStruct((BATCH, SEQ_LEN, D_MODEL), jnp.float32),
        scratch_types=(
            [pltpu.VMEM((BATCH * POS_PER_W,), jnp.int32)]
            + [pltpu.VMEM((CHUNK, D_MODEL), jnp.float32) for _ in range(NBUF)]
            + [pltpu.VMEM((POS_PER_W, D_MODEL), jnp.float32)]
            + [pltpu.SemaphoreType.DMA for _ in range(2 + 2 * NBUF)]
        ),
    )(_body)
    return k(idx, tok_table, pos_table)


def kernel(inputs, tok_table, pos_table):
    return _embed(inputs.astype(jnp.int32), tok_table, pos_table)


# speedup vs baseline: 1.2178x; 1.1749x over previous
import functools
import math

import jax
import jax.numpy as jnp
from jax import lax
from jax.experimental import pallas as pl
from jax.experimental.pallas import tpu as pltpu
from jax.experimental.pallas import tpu_sc as plsc

VOCAB = 100000
SEQ_LEN = 2048
D_MODEL = 768
BATCH = 4

NUM_WORKERS = 32
POS_PER_W = SEQ_LEN // NUM_WORKERS
LANES = 16
GROUPS = D_MODEL // LANES
SCALE = math.sqrt(float(D_MODEL))


def _body(idx_hbm, tok_hbm, pos_hbm, out_hbm, idx_v, tok_v, pos_v, sem):
    wid = lax.axis_index("s") * 2 + lax.axis_index("c")
    pos_base = wid * POS_PER_W

    pltpu.sync_copy(pos_hbm.at[pl.ds(pos_base, POS_PER_W)], pos_v)

    for b in range(BATCH):
        off = b * SEQ_LEN + pos_base
        pltpu.sync_copy(idx_hbm.at[pl.ds(off, POS_PER_W)], idx_v)
        pltpu.async_copy(tok_hbm.at[idx_v], tok_v, sem).wait()

        def row(r, _):
            for j in range(GROUPS):
                sl = pl.ds(j * LANES, LANES)
                tok_v[r, sl] = tok_v[r, sl] * SCALE + pos_v[r, sl]
            return _

        lax.fori_loop(0, POS_PER_W, row, 0)
        pltpu.sync_copy(tok_v, out_hbm.at[pl.ds(off, POS_PER_W)])


@jax.jit
def _embed(idx_flat, tok_table, pos_table):
    mesh = plsc.VectorSubcoreMesh(core_axis_name="c", subcore_axis_name="s")
    k = functools.partial(
        pl.kernel,
        mesh=mesh,
        out_type=jax.ShapeDtypeStruct((BATCH * SEQ_LEN, D_MODEL), jnp.float32),
        scratch_types=[
            pltpu.VMEM((POS_PER_W,), jnp.int32),
            pltpu.VMEM((POS_PER_W, D_MODEL), jnp.float32),
            pltpu.VMEM((POS_PER_W, D_MODEL), jnp.float32),
            pltpu.SemaphoreType.DMA,
        ],
    )(_body)
    return k(idx_flat, tok_table, pos_table)


def kernel(inputs, tok_table, pos_table):
    idx_flat = inputs.astype(jnp.int32).reshape(-1)
    out = _embed(idx_flat, tok_table, pos_table)
    return out.reshape(BATCH, SEQ_LEN, D_MODEL)
